# Initial kernel scaffold; baseline (speedup 1.0000x reference)
import jax
import jax.numpy as jnp
from jax.experimental import pallas as pl


def kernel(x, edge_index, gamma, beta, W1, b1, W2, b2, W3, b3, W4, b4, W5, b5, W6, b6):
    raise NotImplementedError("write your pallas kernel here")


# speedup vs baseline: 2.6769x; 2.6769x over previous
import functools

import jax
import jax.numpy as jnp
from jax import lax
from jax.experimental import pallas as pl
from jax.experimental.pallas import tpu as pltpu
from jax.experimental.pallas import tpu_sc as plsc

N_NODES = 10000
N_PAD = 10240
N_EDGES = 320000
NC, NS = 2, 16
NW = NC * NS
EPW = N_EDGES // NW
CHUNK = 80
NCHUNK = EPW // CHUNK
D = 128

_SC_MESH = dict(core_axis_name="c", subcore_axis_name="s")


def _make_gather():
  mesh = plsc.VectorSubcoreMesh(**_SC_MESH)
  out = jax.ShapeDtypeStruct((N_EDGES, D), jnp.float32)

  @functools.partial(
      pl.kernel, mesh=mesh, out_type=(out, out),
      scratch_types=[
          pltpu.VMEM((CHUNK,), jnp.int32),
          pltpu.VMEM((CHUNK,), jnp.int32),
          pltpu.VMEM((CHUNK, D), jnp.float32),
          pltpu.VMEM((CHUNK, D), jnp.float32),
          pltpu.SemaphoreType.DMA,
          pltpu.SemaphoreType.DMA,
      ])
  def gather_kernel(tab_hbm, dst_hbm, src_hbm, xi_hbm, xj_hbm,
                    di_v, si_v, xi_v, xj_v, sem1, sem2):
    wid = lax.axis_index("s") * NC + lax.axis_index("c")
    base = wid * EPW

    def body(i, carry):
      off = base + i * CHUNK
      pltpu.sync_copy(dst_hbm.at[pl.ds(off, CHUNK)], di_v)
      pltpu.sync_copy(src_hbm.at[pl.ds(off, CHUNK)], si_v)
      cp1 = pltpu.async_copy(tab_hbm.at[di_v], xi_v, sem1)
      cp2 = pltpu.async_copy(tab_hbm.at[si_v], xj_v, sem2)
      cp1.wait()
      cp2.wait()
      pltpu.sync_copy(xi_v, xi_hbm.at[pl.ds(off, CHUNK)])
      pltpu.sync_copy(xj_v, xj_hbm.at[pl.ds(off, CHUNK)])
      return carry

    lax.fori_loop(0, NCHUNK, body, 0)

  return gather_kernel


def _make_scatter():
  mesh = plsc.VectorSubcoreMesh(**_SC_MESH)

  @functools.partial(
      pl.kernel, mesh=mesh,
      out_type=jax.ShapeDtypeStruct((NC, N_PAD, D), jnp.float32),
      scratch_types=[
          pltpu.VMEM((CHUNK,), jnp.int32),
          pltpu.VMEM((CHUNK, D), jnp.float32),
          pltpu.VMEM_SHARED((N_PAD, D), jnp.float32),
      ])
  def scatter_kernel(msg_hbm, dst_hbm, z_hbm, s_hbm, idx_v, msg_v, tab_sh):
    cid = lax.axis_index("c")
    sid = lax.axis_index("s")
    wid = sid * NC + cid
    rows = N_PAD // NS

    pltpu.sync_copy(z_hbm.at[pl.ds(sid * rows, rows)],
                    tab_sh.at[pl.ds(sid * rows, rows)])
    plsc.subcore_barrier()

    base = wid * EPW

    def body(i, carry):
      off = base + i * CHUNK
      pltpu.sync_copy(dst_hbm.at[pl.ds(off, CHUNK)], idx_v)
      pltpu.sync_copy(msg_hbm.at[pl.ds(off, CHUNK)], msg_v)
      pltpu.sync_copy(msg_v, tab_sh.at[idx_v], add=True)
      return carry

    lax.fori_loop(0, NCHUNK, body, 0)
    plsc.subcore_barrier()

    pltpu.sync_copy(tab_sh.at[pl.ds(sid * rows, rows)],
                    s_hbm.at[cid, pl.ds(sid * rows, rows)])

  return scatter_kernel


def _bn_body(x_ref, g_ref, b_ref, o_ref):
  x = x_ref[...]
  mean = jnp.mean(x, axis=0, keepdims=True)
  var = jnp.mean((x - mean) ** 2, axis=0, keepdims=True)
  o_ref[...] = (x - mean) / jnp.sqrt(var + 1e-5) * g_ref[...] + b_ref[...]


def _mlp_body(pad_out, xi_ref, xj_ref, wa_ref, wb_ref, b1_ref, w2_ref,
              b2_ref, w3_ref, b3_ref, o_ref):
  f32 = jnp.float32
  h = jnp.dot(xi_ref[...], wa_ref[...], preferred_element_type=f32)
  h += jnp.dot(xj_ref[...], wb_ref[...], preferred_element_type=f32)
  h = jax.nn.relu(h + b1_ref[...])
  h = jax.nn.relu(jnp.dot(h, w2_ref[...], preferred_element_type=f32)
                  + b2_ref[...])
  h = jnp.dot(h, w3_ref[...], preferred_element_type=f32) + b3_ref[...]
  if pad_out:
    h = jax.nn.relu(h)
    n = h.shape[0]
    pad = jnp.concatenate(
        [jnp.ones((n, 1), f32), jnp.zeros((n, 63), f32)], axis=-1)
    h = jnp.concatenate([h, pad], axis=-1)
  o_ref[...] = h


def _mlp_call(xi, xj, wa, wb, b1, w2, b2, w3, b3, d_mid, pad_out, tile):
  grid = (N_EDGES // tile,)
  full = lambda shape: pl.BlockSpec(shape, lambda i: (0, 0))
  return pl.pallas_call(
      functools.partial(_mlp_body, pad_out),
      grid=grid,
      in_specs=[
          pl.BlockSpec((tile, D), lambda i: (i, 0)),
          pl.BlockSpec((tile, D), lambda i: (i, 0)),
          full((D, 256)), full((D, 256)), full((1, 256)),
          full((256, 256)), full((1, 256)),
          full((256, d_mid)), full((1, d_mid)),
      ],
      out_specs=pl.BlockSpec((tile, D), lambda i: (i, 0)),
      out_shape=jax.ShapeDtypeStruct((N_EDGES, D), jnp.float32),
  )(xi, xj, wa, wb, b1, w2, b2, w3, b3)


def _combine1_body(s_ref, o_ref):
  s = s_ref[0] + s_ref[1]
  cnt = s[:, 64:65]
  inv = 1.0 / jnp.maximum(cnt, 1.0)
  o_ref[...] = s * inv


def _combine2_body(s_ref, s1_ref, o_ref):
  s = s_ref[0] + s_ref[1]
  cnt = s1_ref[0, :, 64:65] + s1_ref[1, :, 64:65]
  inv = 1.0 / jnp.maximum(cnt, 1.0)
  o_ref[...] = (s * inv)[:N_NODES]


def kernel(x, edge_index, gamma, beta, W1, b1, W2, b2, W3, b3,
           W4, b4, W5, b5, W6, b6):
  src = edge_index[0].astype(jnp.int32)
  dst = edge_index[1].astype(jnp.int32)
  f32 = jnp.float32

  w1a = W1[:128] - W1[128:]
  w1b = W1[128:]
  zw = jnp.zeros((64, 256), f32)
  w4a = jnp.concatenate([W4[:64] - W4[64:], zw], axis=0)
  w4b = jnp.concatenate([W4[64:], zw], axis=0)
  b1r, b2r, b3r = b1[None, :], b2[None, :], b3[None, :]
  b4r, b5r, b6r = b4[None, :], b5[None, :], b6[None, :]
  z128 = jnp.zeros((N_PAD, D), f32)

  h = pl.pallas_call(
      _bn_body,
      out_shape=jax.ShapeDtypeStruct((N_NODES, D), f32),
  )(x, gamma[None, :], beta[None, :])

  gather = _make_gather()
  scatter = _make_scatter()

  xi, xj = gather(h, dst, src)
  m1 = _mlp_call(xi, xj, w1a, w1b, b1r, W2, b2r, W3, b3r,
                 d_mid=64, pad_out=True, tile=2000)
  s1 = scatter(m1, dst, z128)
  h1 = pl.pallas_call(
      _combine1_body,
      out_shape=jax.ShapeDtypeStruct((N_PAD, D), f32),
  )(s1)

  yi, yj = gather(h1, dst, src)
  m2 = _mlp_call(yi, yj, w4a, w4b, b4r, W5, b5r, W6, b6r,
                 d_mid=128, pad_out=False, tile=2000)
  s2 = scatter(m2, dst, z128)
  out = pl.pallas_call(
      _combine2_body,
      out_shape=jax.ShapeDtypeStruct((N_NODES, D), f32),
  )(s2, s1)
  return out


# speedup vs baseline: 4.4969x; 1.6799x over previous
import functools

import jax
import jax.numpy as jnp
from jax import lax
from jax.experimental import pallas as pl
from jax.experimental.pallas import tpu as pltpu
from jax.experimental.pallas import tpu_sc as plsc

N_NODES = 10000
N_PAD = 10240
N_EDGES = 320000
NC, NS = 2, 16
NW = NC * NS
CHUNK = 80
D = 128
E_SLICES = (0, 160000), (160000, 160000)

_SC_MESH = dict(core_axis_name="c", subcore_axis_name="s")


def _make_gather(start, ne):
  epw = ne // NW
  nfull = epw // CHUNK
  tail = epw - nfull * CHUNK
  mesh = plsc.VectorSubcoreMesh(**_SC_MESH)
  out = jax.ShapeDtypeStruct((ne, D), jnp.float32)

  ngrp = nfull // 2
  assert ngrp * 2 == nfull

  @functools.partial(
      pl.kernel, mesh=mesh, out_type=(out, out),
      scratch_types=[
          pltpu.VMEM((epw,), jnp.int32),
          pltpu.VMEM((epw,), jnp.int32),
          pltpu.VMEM((CHUNK, D), jnp.float32),
          pltpu.VMEM((CHUNK, D), jnp.float32),
          pltpu.VMEM((CHUNK, D), jnp.float32),
          pltpu.VMEM((CHUNK, D), jnp.float32),
          pltpu.SemaphoreType.DMA,
          pltpu.SemaphoreType.DMA,
          pltpu.SemaphoreType.DMA,
          pltpu.SemaphoreType.DMA,
      ])
  def gather_kernel(tab_hbm, dst_hbm, src_hbm, xi_hbm, xj_hbm,
                    di_v, si_v, xi0, xj0, xi1, xj1, gs0, gs1, ws0, ws1):
    wid = lax.axis_index("s") * NC + lax.axis_index("c")
    gbase = start + wid * epw
    obase = wid * epw
    pltpu.sync_copy(dst_hbm.at[pl.ds(gbase, epw)], di_v)
    pltpu.sync_copy(src_hbm.at[pl.ds(gbase, epw)], si_v)
    xi_b, xj_b = (xi0, xi1), (xj0, xj1)
    gs, ws = (gs0, gs1), (ws0, ws1)

    def start_g(c, p):
      o = c * CHUNK
      pltpu.async_copy(tab_hbm.at[di_v.at[pl.ds(o, CHUNK)]], xi_b[p], gs[p])
      pltpu.async_copy(tab_hbm.at[si_v.at[pl.ds(o, CHUNK)]], xj_b[p], gs[p])

    def wait_g(p):
      pltpu.make_async_copy(tab_hbm.at[di_v.at[pl.ds(0, CHUNK)]],
                            xi_b[p], gs[p]).wait()
      pltpu.make_async_copy(tab_hbm.at[di_v.at[pl.ds(0, CHUNK)]],
                            xj_b[p], gs[p]).wait()

    def start_wb(c, p):
      o = obase + c * CHUNK
      pltpu.async_copy(xi_b[p], xi_hbm.at[pl.ds(o, CHUNK)], ws[p])
      pltpu.async_copy(xj_b[p], xj_hbm.at[pl.ds(o, CHUNK)], ws[p])

    def wait_wb(p):
      pltpu.make_async_copy(xi_b[p], xi_hbm.at[pl.ds(obase, CHUNK)],
                            ws[p]).wait()
      pltpu.make_async_copy(xj_b[p], xj_hbm.at[pl.ds(obase, CHUNK)],
                            ws[p]).wait()

    start_g(0, 0)

    def body(g, carry):
      c0 = 2 * g

      @pl.when(g > 0)
      def _():
        wait_wb(1)

      start_g(c0 + 1, 1)
      wait_g(0)
      start_wb(c0, 0)
      wait_wb(0)

      @pl.when(g < ngrp - 1)
      def _():
        start_g(c0 + 2, 0)

      wait_g(1)
      start_wb(c0 + 1, 1)
      return carry

    lax.fori_loop(0, ngrp, body, 0)
    wait_wb(1)
    if tail:
      o = nfull * CHUNK
      cp1 = pltpu.async_copy(tab_hbm.at[di_v.at[pl.ds(o, tail)]],
                             xi0.at[pl.ds(0, tail)], gs0)
      cp2 = pltpu.async_copy(tab_hbm.at[si_v.at[pl.ds(o, tail)]],
                             xj0.at[pl.ds(0, tail)], gs0)
      cp1.wait()
      cp2.wait()
      pltpu.sync_copy(xi0.at[pl.ds(0, tail)],
                      xi_hbm.at[pl.ds(obase + o, tail)])
      pltpu.sync_copy(xj0.at[pl.ds(0, tail)],
                      xj_hbm.at[pl.ds(obase + o, tail)])

  return gather_kernel


def _make_scatter(start, ne):
  epw = ne // NW
  nfull = epw // CHUNK
  tail = epw - nfull * CHUNK
  mesh = plsc.VectorSubcoreMesh(**_SC_MESH)

  @functools.partial(
      pl.kernel, mesh=mesh,
      out_type=jax.ShapeDtypeStruct((NC, N_PAD, D), jnp.float32),
      scratch_types=[
          pltpu.VMEM((CHUNK,), jnp.int32),
          pltpu.VMEM((CHUNK,), jnp.int32),
          pltpu.VMEM((CHUNK, D), jnp.float32),
          pltpu.VMEM((CHUNK, D), jnp.float32),
          pltpu.VMEM_SHARED((N_PAD, D), jnp.float32),
          pltpu.SemaphoreType.DMA,
          pltpu.SemaphoreType.DMA,
      ] + ([pltpu.VMEM((tail,), jnp.int32)] if tail else []))
  def scatter_kernel(msg_hbm, dst_hbm, z_hbm, s_hbm, idx0, idx1, msg0, msg1,
                     tab_sh, ls0, ls1, *tail_refs):
    cid = lax.axis_index("c")
    sid = lax.axis_index("s")
    wid = sid * NC + cid
    rows = N_PAD // NS

    pltpu.sync_copy(z_hbm.at[pl.ds(sid * rows, rows)],
                    tab_sh.at[pl.ds(sid * rows, rows)])
    plsc.subcore_barrier()
    ibase = start + wid * epw
    mbase = wid * epw
    idx_b, msg_b, ls = (idx0, idx1), (msg0, msg1), (ls0, ls1)
    ngrp = nfull // 2

    def start_ld(c, p):
      o = c * CHUNK
      pltpu.async_copy(dst_hbm.at[pl.ds(ibase + o, CHUNK)], idx_b[p], ls[p])
      pltpu.async_copy(msg_hbm.at[pl.ds(mbase + o, CHUNK)], msg_b[p], ls[p])

    def wait_ld(p):
      pltpu.make_async_copy(dst_hbm.at[pl.ds(ibase, CHUNK)],
                            idx_b[p], ls[p]).wait()
      pltpu.make_async_copy(msg_hbm.at[pl.ds(mbase, CHUNK)],
                            msg_b[p], ls[p]).wait()

    start_ld(0, 0)

    def body(g, carry):
      c0 = 2 * g
      start_ld(c0 + 1, 1)
      wait_ld(0)
      pltpu.sync_copy(msg0, tab_sh.at[idx0], add=True)

      @pl.when(g < ngrp - 1)
      def _():
        start_ld(c0 + 2, 0)

      wait_ld(1)
      pltpu.sync_copy(msg1, tab_sh.at[idx1], add=True)
      return carry

    lax.fori_loop(0, ngrp, body, 0)
    if tail:
      idx_t = tail_refs[0]
      o = nfull * CHUNK
      pltpu.sync_copy(dst_hbm.at[pl.ds(ibase + o, tail)], idx_t)
      pltpu.sync_copy(msg_hbm.at[pl.ds(mbase + o, tail)],
                      msg0.at[pl.ds(0, tail)])
      pltpu.sync_copy(msg0.at[pl.ds(0, tail)], tab_sh.at[idx_t], add=True)
    plsc.subcore_barrier()
    pltpu.sync_copy(tab_sh.at[pl.ds(sid * rows, rows)],
                    s_hbm.at[cid, pl.ds(sid * rows, rows)])

  return scatter_kernel


def _bn_body(x_ref, g_ref, b_ref, o_ref):
  x = x_ref[...]
  mean = jnp.mean(x, axis=0, keepdims=True)
  var = jnp.mean((x - mean) ** 2, axis=0, keepdims=True)
  o_ref[...] = (x - mean) / jnp.sqrt(var + 1e-5) * g_ref[...] + b_ref[...]


def _mlp_body(pad_out, xi_ref, xj_ref, wa_ref, wb_ref, b1_ref, w2_ref,
              b2_ref, w3_ref, b3_ref, o_ref):
  f32, bf16 = jnp.float32, jnp.bfloat16
  h = jnp.dot(xi_ref[...].astype(bf16), wa_ref[...],
              preferred_element_type=f32)
  h += jnp.dot(xj_ref[...].astype(bf16), wb_ref[...],
               preferred_element_type=f32)
  h = jax.nn.relu(h + b1_ref[...])
  h = jax.nn.relu(jnp.dot(h.astype(bf16), w2_ref[...],
                          preferred_element_type=f32) + b2_ref[...])
  h = jnp.dot(h.astype(bf16), w3_ref[...],
              preferred_element_type=f32) + b3_ref[...]
  if pad_out:
    h = jax.nn.relu(h)
    n = h.shape[0]
    pad = jnp.concatenate(
        [jnp.ones((n, 1), f32), jnp.zeros((n, 63), f32)], axis=-1)
    h = jnp.concatenate([h, pad], axis=-1)
  o_ref[...] = h


def _mlp_call(xi, xj, wa, wb, b1, w2, b2, w3, b3, d_mid, pad_out, tile):
  ne = xi.shape[0]
  grid = (ne // tile,)
  full = lambda shape: pl.BlockSpec(shape, lambda i: (0, 0))
  return pl.pallas_call(
      functools.partial(_mlp_body, pad_out),
      grid=grid,
      in_specs=[
          pl.BlockSpec((tile, D), lambda i: (i, 0)),
          pl.BlockSpec((tile, D), lambda i: (i, 0)),
          full((D, 256)), full((D, 256)), full((1, 256)),
          full((256, 256)), full((1, 256)),
          full((256, d_mid)), full((1, d_mid)),
      ],
      out_specs=pl.BlockSpec((tile, D), lambda i: (i, 0)),
      out_shape=jax.ShapeDtypeStruct((ne, D), jnp.float32),
  )(xi, xj, wa, wb, b1, w2, b2, w3, b3)


def _combine1_body(sa_ref, sb_ref, o_ref):
  s = sa_ref[0] + sa_ref[1] + sb_ref[0] + sb_ref[1]
  cnt = s[:, 64:65]
  inv = 1.0 / jnp.maximum(cnt, 1.0)
  o_ref[...] = s * inv


def _combine2_body(sa_ref, sb_ref, ca_ref, cb_ref, o_ref):
  s = sa_ref[0] + sa_ref[1] + sb_ref[0] + sb_ref[1]
  cnt = (ca_ref[0, :, 64:65] + ca_ref[1, :, 64:65]
         + cb_ref[0, :, 64:65] + cb_ref[1, :, 64:65])
  inv = 1.0 / jnp.maximum(cnt, 1.0)
  o_ref[...] = (s * inv)[:N_NODES]


def kernel(x, edge_index, gamma, beta, W1, b1, W2, b2, W3, b3,
           W4, b4, W5, b5, W6, b6):
  src = edge_index[0].astype(jnp.int32)
  dst = edge_index[1].astype(jnp.int32)
  f32, bf16 = jnp.float32, jnp.bfloat16

  w1a = (W1[:128] - W1[128:]).astype(bf16)
  w1b = W1[128:].astype(bf16)
  zw = jnp.zeros((64, 256), f32)
  w4a = jnp.concatenate([W4[:64] - W4[64:], zw], axis=0).astype(bf16)
  w4b = jnp.concatenate([W4[64:], zw], axis=0).astype(bf16)
  w2c, w3c = W2.astype(bf16), W3.astype(bf16)
  w5c, w6c = W5.astype(bf16), W6.astype(bf16)
  b1r, b2r, b3r = b1[None, :], b2[None, :], b3[None, :]
  b4r, b5r, b6r = b4[None, :], b5[None, :], b6[None, :]
  z128 = jnp.zeros((N_PAD, D), f32)

  h = pl.pallas_call(
      _bn_body,
      out_shape=jax.ShapeDtypeStruct((N_NODES, D), f32),
  )(x, gamma[None, :], beta[None, :])

  gathers = [_make_gather(s, n) for s, n in E_SLICES]
  scatters = [_make_scatter(s, n) for s, n in E_SLICES]

  def layer(tab, wa, wb, bb1, w2, bb2, w3, bb3, d_mid, pad_out):
    parts = []
    pairs = [g(tab, dst, src) for g in gathers]
    for (xi, xj), sc in zip(pairs, scatters):
      m = _mlp_call(xi, xj, wa, wb, bb1, w2, bb2, w3, bb3,
                    d_mid=d_mid, pad_out=pad_out, tile=2000)
      parts.append(sc(m, dst, z128))
    return parts

  s1a, s1b = layer(h, w1a, w1b, b1r, w2c, b2r, w3c, b3r, 64, True)
  h1 = pl.pallas_call(
      _combine1_body,
      out_shape=jax.ShapeDtypeStruct((N_PAD, D), f32),
  )(s1a, s1b)

  s2a, s2b = layer(h1, w4a, w4b, b4r, w5c, b5r, w6c, b6r, 128, False)
  out = pl.pallas_call(
      _combine2_body,
      out_shape=jax.ShapeDtypeStruct((N_NODES, D), f32),
  )(s2a, s2b, s1a, s1b)
  return out


# speedup vs baseline: 5.1069x; 1.1357x over previous
import functools

import jax
import jax.numpy as jnp
from jax import lax
from jax.experimental import pallas as pl
from jax.experimental.pallas import tpu as pltpu
from jax.experimental.pallas import tpu_sc as plsc

N_NODES = 10000
N_PAD = 10240
N_EDGES = 320000
NC, NS = 2, 16
NW = NC * NS
CHUNK = 80
D = 128
E_SLICES = (0, 160000), (160000, 160000)

_SC_MESH = dict(core_axis_name="c", subcore_axis_name="s")


def _make_gather(start, ne, n_tab):
  epw = ne // NW
  nfull = epw // CHUNK
  tail = epw - nfull * CHUNK
  mesh = plsc.VectorSubcoreMesh(**_SC_MESH)
  out = jax.ShapeDtypeStruct((ne, D), jnp.float32)
  trows = n_tab // NS

  ngrp = nfull // 2
  assert ngrp * 2 == nfull

  @functools.partial(
      pl.kernel, mesh=mesh, out_type=(out, out),
      scratch_types=[
          pltpu.VMEM((CHUNK,), jnp.int32),
          pltpu.VMEM((CHUNK,), jnp.int32),
          pltpu.VMEM((CHUNK,), jnp.int32),
          pltpu.VMEM((CHUNK,), jnp.int32),
          pltpu.VMEM((CHUNK, D), jnp.float32),
          pltpu.VMEM((CHUNK, D), jnp.float32),
          pltpu.VMEM((CHUNK, D), jnp.float32),
          pltpu.VMEM((CHUNK, D), jnp.float32),
          pltpu.VMEM_SHARED((n_tab, D), jnp.float32),
          pltpu.SemaphoreType.DMA,
          pltpu.SemaphoreType.DMA,
          pltpu.SemaphoreType.DMA,
          pltpu.SemaphoreType.DMA,
          pltpu.SemaphoreType.DMA,
          pltpu.SemaphoreType.DMA,
      ])
  def gather_kernel(tab_hbm, dst_hbm, src_hbm, xi_hbm, xj_hbm,
                    di0, si0, di1, si1, xi0, xj0, xi1, xj1, tab_sh,
                    is0, is1, gs0, gs1, ws0, ws1):
    sid = lax.axis_index("s")
    wid = sid * NC + lax.axis_index("c")
    gbase = start + wid * epw
    obase = wid * epw
    pltpu.sync_copy(tab_hbm.at[pl.ds(sid * trows, trows)],
                    tab_sh.at[pl.ds(sid * trows, trows)])
    plsc.subcore_barrier()
    di_b, si_b = (di0, di1), (si0, si1)
    xi_b, xj_b = (xi0, xi1), (xj0, xj1)
    isem, gs, ws = (is0, is1), (gs0, gs1), (ws0, ws1)

    def start_idx(c, p):
      o = gbase + c * CHUNK
      pltpu.async_copy(dst_hbm.at[pl.ds(o, CHUNK)], di_b[p], isem[p])
      pltpu.async_copy(src_hbm.at[pl.ds(o, CHUNK)], si_b[p], isem[p])

    def wait_idx(p):
      pltpu.make_async_copy(dst_hbm.at[pl.ds(gbase, CHUNK)],
                            di_b[p], isem[p]).wait()
      pltpu.make_async_copy(dst_hbm.at[pl.ds(gbase, CHUNK)],
                            si_b[p], isem[p]).wait()

    def start_g(p):
      pltpu.async_copy(tab_sh.at[di_b[p]], xi_b[p], gs[p])
      pltpu.async_copy(tab_sh.at[si_b[p]], xj_b[p], gs[p])

    def wait_g(p):
      pltpu.make_async_copy(tab_sh.at[di_b[p]], xi_b[p], gs[p]).wait()
      pltpu.make_async_copy(tab_sh.at[di_b[p]], xj_b[p], gs[p]).wait()

    def start_wb(c, p):
      o = obase + c * CHUNK
      pltpu.async_copy(xi_b[p], xi_hbm.at[pl.ds(o, CHUNK)], ws[p])
      pltpu.async_copy(xj_b[p], xj_hbm.at[pl.ds(o, CHUNK)], ws[p])

    def wait_wb(p):
      pltpu.make_async_copy(xi_b[p], xi_hbm.at[pl.ds(obase, CHUNK)],
                            ws[p]).wait()
      pltpu.make_async_copy(xj_b[p], xj_hbm.at[pl.ds(obase, CHUNK)],
                            ws[p]).wait()

    start_idx(0, 0)

    def body(g, carry):
      c0 = 2 * g
      wait_idx(0)

      @pl.when(g > 0)
      def _():
        wait_wb(0)

      start_g(0)

      @pl.when(g > 0)
      def _():
        wait_wb(1)

      start_idx(c0 + 1, 1)
      wait_g(0)
      start_wb(c0, 0)

      @pl.when(g < ngrp - 1)
      def _():
        start_idx(c0 + 2, 0)

      wait_idx(1)
      start_g(1)
      wait_g(1)
      start_wb(c0 + 1, 1)
      return carry

    lax.fori_loop(0, ngrp, body, 0)
    wait_wb(0)
    wait_wb(1)
    if tail:
      o = nfull * CHUNK
      pltpu.sync_copy(dst_hbm.at[pl.ds(gbase + o, tail)],
                      di0.at[pl.ds(0, tail)])
      pltpu.sync_copy(src_hbm.at[pl.ds(gbase + o, tail)],
                      si0.at[pl.ds(0, tail)])
      cp1 = pltpu.async_copy(tab_sh.at[di0.at[pl.ds(0, tail)]],
                             xi0.at[pl.ds(0, tail)], gs0)
      cp2 = pltpu.async_copy(tab_sh.at[si0.at[pl.ds(0, tail)]],
                             xj0.at[pl.ds(0, tail)], gs0)
      cp1.wait()
      cp2.wait()
      pltpu.sync_copy(xi0.at[pl.ds(0, tail)],
                      xi_hbm.at[pl.ds(obase + o, tail)])
      pltpu.sync_copy(xj0.at[pl.ds(0, tail)],
                      xj_hbm.at[pl.ds(obase + o, tail)])

  return gather_kernel


def _make_scatter(start, ne):
  epw = ne // NW
  nfull = epw // CHUNK
  tail = epw - nfull * CHUNK
  mesh = plsc.VectorSubcoreMesh(**_SC_MESH)

  @functools.partial(
      pl.kernel, mesh=mesh,
      out_type=jax.ShapeDtypeStruct((NC, N_PAD, D), jnp.float32),
      scratch_types=[
          pltpu.VMEM((CHUNK,), jnp.int32),
          pltpu.VMEM((CHUNK,), jnp.int32),
          pltpu.VMEM((CHUNK, D), jnp.float32),
          pltpu.VMEM((CHUNK, D), jnp.float32),
          pltpu.VMEM_SHARED((N_PAD, D), jnp.float32),
          pltpu.SemaphoreType.DMA,
          pltpu.SemaphoreType.DMA,
      ] + ([pltpu.VMEM((tail,), jnp.int32)] if tail else []))
  def scatter_kernel(msg_hbm, dst_hbm, z_hbm, s_hbm, idx0, idx1, msg0, msg1,
                     tab_sh, ls0, ls1, *tail_refs):
    cid = lax.axis_index("c")
    sid = lax.axis_index("s")
    wid = sid * NC + cid
    rows = N_PAD // NS

    pltpu.sync_copy(z_hbm.at[pl.ds(sid * rows, rows)],
                    tab_sh.at[pl.ds(sid * rows, rows)])
    plsc.subcore_barrier()
    ibase = start + wid * epw
    mbase = wid * epw
    idx_b, msg_b, ls = (idx0, idx1), (msg0, msg1), (ls0, ls1)
    ngrp = nfull // 2

    def start_ld(c, p):
      o = c * CHUNK
      pltpu.async_copy(dst_hbm.at[pl.ds(ibase + o, CHUNK)], idx_b[p], ls[p])
      pltpu.async_copy(msg_hbm.at[pl.ds(mbase + o, CHUNK)], msg_b[p], ls[p])

    def wait_ld(p):
      pltpu.make_async_copy(dst_hbm.at[pl.ds(ibase, CHUNK)],
                            idx_b[p], ls[p]).wait()
      pltpu.make_async_copy(msg_hbm.at[pl.ds(mbase, CHUNK)],
                            msg_b[p], ls[p]).wait()

    start_ld(0, 0)

    def body(g, carry):
      c0 = 2 * g
      start_ld(c0 + 1, 1)
      wait_ld(0)
      pltpu.sync_copy(msg0, tab_sh.at[idx0], add=True)

      @pl.when(g < ngrp - 1)
      def _():
        start_ld(c0 + 2, 0)

      wait_ld(1)
      pltpu.sync_copy(msg1, tab_sh.at[idx1], add=True)
      return carry

    lax.fori_loop(0, ngrp, body, 0)
    if tail:
      idx_t = tail_refs[0]
      o = nfull * CHUNK
      pltpu.sync_copy(dst_hbm.at[pl.ds(ibase + o, tail)], idx_t)
      pltpu.sync_copy(msg_hbm.at[pl.ds(mbase + o, tail)],
                      msg0.at[pl.ds(0, tail)])
      pltpu.sync_copy(msg0.at[pl.ds(0, tail)], tab_sh.at[idx_t], add=True)
    plsc.subcore_barrier()
    pltpu.sync_copy(tab_sh.at[pl.ds(sid * rows, rows)],
                    s_hbm.at[cid, pl.ds(sid * rows, rows)])

  return scatter_kernel


def _bn_body(x_ref, g_ref, b_ref, o_ref):
  x = x_ref[...]
  mean = jnp.mean(x, axis=0, keepdims=True)
  var = jnp.mean((x - mean) ** 2, axis=0, keepdims=True)
  o_ref[pl.ds(0, N_NODES), :] = (
      (x - mean) / jnp.sqrt(var + 1e-5) * g_ref[...] + b_ref[...])


def _mlp_body(pad_out, xi_ref, xj_ref, wa_ref, wb_ref, b1_ref, w2_ref,
              b2_ref, w3_ref, b3_ref, o_ref):
  f32, bf16 = jnp.float32, jnp.bfloat16
  h = jnp.dot(xi_ref[...].astype(bf16), wa_ref[...],
              preferred_element_type=f32)
  h += jnp.dot(xj_ref[...].astype(bf16), wb_ref[...],
               preferred_element_type=f32)
  h = jax.nn.relu(h + b1_ref[...])
  h = jax.nn.relu(jnp.dot(h.astype(bf16), w2_ref[...],
                          preferred_element_type=f32) + b2_ref[...])
  h = jnp.dot(h.astype(bf16), w3_ref[...],
              preferred_element_type=f32) + b3_ref[...]
  if pad_out:
    h = jax.nn.relu(h)
    n = h.shape[0]
    pad = jnp.concatenate(
        [jnp.ones((n, 1), f32), jnp.zeros((n, 63), f32)], axis=-1)
    h = jnp.concatenate([h, pad], axis=-1)
  o_ref[...] = h


def _mlp_call(xi, xj, wa, wb, b1, w2, b2, w3, b3, d_mid, pad_out, tile):
  ne = xi.shape[0]
  grid = (ne // tile,)
  full = lambda shape: pl.BlockSpec(shape, lambda i: (0, 0))
  return pl.pallas_call(
      functools.partial(_mlp_body, pad_out),
      grid=grid,
      in_specs=[
          pl.BlockSpec((tile, D), lambda i: (i, 0)),
          pl.BlockSpec((tile, D), lambda i: (i, 0)),
          full((D, 256)), full((D, 256)), full((1, 256)),
          full((256, 256)), full((1, 256)),
          full((256, d_mid)), full((1, d_mid)),
      ],
      out_specs=pl.BlockSpec((tile, D), lambda i: (i, 0)),
      out_shape=jax.ShapeDtypeStruct((ne, D), jnp.float32),
  )(xi, xj, wa, wb, b1, w2, b2, w3, b3)


def _combine1_body(sa_ref, sb_ref, o_ref):
  s = sa_ref[0] + sa_ref[1] + sb_ref[0] + sb_ref[1]
  cnt = s[:, 64:65]
  inv = 1.0 / jnp.maximum(cnt, 1.0)
  o_ref[...] = s * inv


def _combine2_body(sa_ref, sb_ref, ca_ref, cb_ref, o_ref):
  s = sa_ref[0] + sa_ref[1] + sb_ref[0] + sb_ref[1]
  cnt = (ca_ref[0, :, 64:65] + ca_ref[1, :, 64:65]
         + cb_ref[0, :, 64:65] + cb_ref[1, :, 64:65])
  inv = 1.0 / jnp.maximum(cnt, 1.0)
  o_ref[...] = (s * inv)[:N_NODES]


def kernel(x, edge_index, gamma, beta, W1, b1, W2, b2, W3, b3,
           W4, b4, W5, b5, W6, b6):
  src = edge_index[0].astype(jnp.int32)
  dst = edge_index[1].astype(jnp.int32)
  f32, bf16 = jnp.float32, jnp.bfloat16

  w1a = (W1[:128] - W1[128:]).astype(bf16)
  w1b = W1[128:].astype(bf16)
  zw = jnp.zeros((64, 256), f32)
  w4a = jnp.concatenate([W4[:64] - W4[64:], zw], axis=0).astype(bf16)
  w4b = jnp.concatenate([W4[64:], zw], axis=0).astype(bf16)
  w2c, w3c = W2.astype(bf16), W3.astype(bf16)
  w5c, w6c = W5.astype(bf16), W6.astype(bf16)
  b1r, b2r, b3r = b1[None, :], b2[None, :], b3[None, :]
  b4r, b5r, b6r = b4[None, :], b5[None, :], b6[None, :]
  z128 = jnp.zeros((N_PAD, D), f32)

  h = pl.pallas_call(
      _bn_body,
      out_shape=jax.ShapeDtypeStruct((N_PAD, D), f32),
  )(x, gamma[None, :], beta[None, :])

  scatters = [_make_scatter(s, n) for s, n in E_SLICES]

  def layer(tab, wa, wb, bb1, w2, bb2, w3, bb3, d_mid, pad_out):
    parts = []
    gathers = [_make_gather(s, n, N_PAD) for s, n in E_SLICES]
    pairs = [g(tab, dst, src) for g in gathers]
    for (xi, xj), sc in zip(pairs, scatters):
      m = _mlp_call(xi, xj, wa, wb, bb1, w2, bb2, w3, bb3,
                    d_mid=d_mid, pad_out=pad_out, tile=2000)
      parts.append(sc(m, dst, z128))
    return parts

  s1a, s1b = layer(h, w1a, w1b, b1r, w2c, b2r, w3c, b3r, 64, True)
  h1 = pl.pallas_call(
      _combine1_body,
      out_shape=jax.ShapeDtypeStruct((N_PAD, D), f32),
  )(s1a, s1b)

  s2a, s2b = layer(h1, w4a, w4b, b4r, w5c, b5r, w6c, b6r, 128, False)
  out = pl.pallas_call(
      _combine2_body,
      out_shape=jax.ShapeDtypeStruct((N_NODES, D), f32),
  )(s2a, s2b, s1a, s1b)
  return out


# speedup vs baseline: 5.1657x; 1.0115x over previous
import functools

import jax
import jax.numpy as jnp
from jax import lax
from jax.experimental import pallas as pl
from jax.experimental.pallas import tpu as pltpu
from jax.experimental.pallas import tpu_sc as plsc

N_NODES = 10000
N_PAD = 10240
N_EDGES = 320000
NC, NS = 2, 16
NW = NC * NS
CHUNK = 80
D = 128
E_SLICES = (0, 160000), (160000, 160000)

_SC_MESH = dict(core_axis_name="c", subcore_axis_name="s")


def _make_gather(start, ne, n_tab):
  epw = ne // NW
  nfull = epw // CHUNK
  tail = epw - nfull * CHUNK
  mesh = plsc.VectorSubcoreMesh(**_SC_MESH)
  out = jax.ShapeDtypeStruct((ne, D), jnp.float32)
  trows = n_tab // NS

  ngrp = nfull // 2
  assert ngrp * 2 == nfull

  @functools.partial(
      pl.kernel, mesh=mesh, out_type=(out, out),
      scratch_types=[
          pltpu.VMEM((CHUNK,), jnp.int32),
          pltpu.VMEM((CHUNK,), jnp.int32),
          pltpu.VMEM((CHUNK,), jnp.int32),
          pltpu.VMEM((CHUNK,), jnp.int32),
          pltpu.VMEM((CHUNK, D), jnp.float32),
          pltpu.VMEM((CHUNK, D), jnp.float32),
          pltpu.VMEM((CHUNK, D), jnp.float32),
          pltpu.VMEM((CHUNK, D), jnp.float32),
          pltpu.VMEM_SHARED((n_tab, D), jnp.float32),
          pltpu.SemaphoreType.DMA,
          pltpu.SemaphoreType.DMA,
          pltpu.SemaphoreType.DMA,
          pltpu.SemaphoreType.DMA,
          pltpu.SemaphoreType.DMA,
          pltpu.SemaphoreType.DMA,
      ])
  def gather_kernel(tab_hbm, dst_hbm, src_hbm, xi_hbm, xj_hbm,
                    di0, si0, di1, si1, xi0, xj0, xi1, xj1, tab_sh,
                    is0, is1, gs0, gs1, ws0, ws1):
    sid = lax.axis_index("s")
    wid = sid * NC + lax.axis_index("c")
    gbase = start + wid * epw
    obase = wid * epw
    pltpu.sync_copy(tab_hbm.at[pl.ds(sid * trows, trows)],
                    tab_sh.at[pl.ds(sid * trows, trows)])
    plsc.subcore_barrier()
    di_b, si_b = (di0, di1), (si0, si1)
    xi_b, xj_b = (xi0, xi1), (xj0, xj1)
    isem, gs, ws = (is0, is1), (gs0, gs1), (ws0, ws1)

    def start_idx(c, p):
      o = gbase + c * CHUNK
      pltpu.async_copy(dst_hbm.at[pl.ds(o, CHUNK)], di_b[p], isem[p])
      pltpu.async_copy(src_hbm.at[pl.ds(o, CHUNK)], si_b[p], isem[p])

    def wait_idx(p):
      pltpu.make_async_copy(dst_hbm.at[pl.ds(gbase, CHUNK)],
                            di_b[p], isem[p]).wait()
      pltpu.make_async_copy(dst_hbm.at[pl.ds(gbase, CHUNK)],
                            si_b[p], isem[p]).wait()

    def start_g(p):
      pltpu.async_copy(tab_sh.at[di_b[p]], xi_b[p], gs[p])
      pltpu.async_copy(tab_sh.at[si_b[p]], xj_b[p], gs[p])

    def wait_g(p):
      pltpu.make_async_copy(tab_sh.at[di_b[p]], xi_b[p], gs[p]).wait()
      pltpu.make_async_copy(tab_sh.at[di_b[p]], xj_b[p], gs[p]).wait()

    def start_wb(c, p):
      o = obase + c * CHUNK
      pltpu.async_copy(xi_b[p], xi_hbm.at[pl.ds(o, CHUNK)], ws[p])
      pltpu.async_copy(xj_b[p], xj_hbm.at[pl.ds(o, CHUNK)], ws[p])

    def wait_wb(p):
      pltpu.make_async_copy(xi_b[p], xi_hbm.at[pl.ds(obase, CHUNK)],
                            ws[p]).wait()
      pltpu.make_async_copy(xj_b[p], xj_hbm.at[pl.ds(obase, CHUNK)],
                            ws[p]).wait()

    start_idx(0, 0)

    def body(g, carry):
      c0 = 2 * g
      wait_idx(0)

      @pl.when(g > 0)
      def _():
        wait_wb(0)

      start_g(0)

      @pl.when(g > 0)
      def _():
        wait_wb(1)

      start_idx(c0 + 1, 1)
      wait_g(0)
      start_wb(c0, 0)

      @pl.when(g < ngrp - 1)
      def _():
        start_idx(c0 + 2, 0)

      wait_idx(1)
      start_g(1)
      wait_g(1)
      start_wb(c0 + 1, 1)
      return carry

    lax.fori_loop(0, ngrp, body, 0)
    wait_wb(0)
    wait_wb(1)
    if tail:
      o = nfull * CHUNK
      pltpu.sync_copy(dst_hbm.at[pl.ds(gbase + o, tail)],
                      di0.at[pl.ds(0, tail)])
      pltpu.sync_copy(src_hbm.at[pl.ds(gbase + o, tail)],
                      si0.at[pl.ds(0, tail)])
      cp1 = pltpu.async_copy(tab_sh.at[di0.at[pl.ds(0, tail)]],
                             xi0.at[pl.ds(0, tail)], gs0)
      cp2 = pltpu.async_copy(tab_sh.at[si0.at[pl.ds(0, tail)]],
                             xj0.at[pl.ds(0, tail)], gs0)
      cp1.wait()
      cp2.wait()
      pltpu.sync_copy(xi0.at[pl.ds(0, tail)],
                      xi_hbm.at[pl.ds(obase + o, tail)])
      pltpu.sync_copy(xj0.at[pl.ds(0, tail)],
                      xj_hbm.at[pl.ds(obase + o, tail)])

  return gather_kernel


def _make_scatter(start, ne):
  epw = ne // NW
  nfull = epw // CHUNK
  tail = epw - nfull * CHUNK
  mesh = plsc.VectorSubcoreMesh(**_SC_MESH)
  NB = 4
  ngrp = nfull // NB
  rem = nfull - NB * ngrp

  @functools.partial(
      pl.kernel, mesh=mesh,
      out_type=jax.ShapeDtypeStruct((NC, N_PAD, D), jnp.float32),
      scratch_types=(
          [pltpu.VMEM((CHUNK,), jnp.int32) for _ in range(NB)]
          + [pltpu.VMEM((CHUNK, D), jnp.float32) for _ in range(NB)]
          + [pltpu.VMEM_SHARED((N_PAD, D), jnp.float32)]
          + [pltpu.SemaphoreType.DMA for _ in range(NB)]
          + ([pltpu.VMEM((tail,), jnp.int32)] if tail else [])))
  def scatter_kernel(msg_hbm, dst_hbm, z_hbm, s_hbm, *refs):
    idx_b = refs[:NB]
    msg_b = refs[NB:2 * NB]
    tab_sh = refs[2 * NB]
    ls = refs[2 * NB + 1:3 * NB + 1]
    tail_refs = refs[3 * NB + 1:]
    cid = lax.axis_index("c")
    sid = lax.axis_index("s")
    wid = sid * NC + cid
    rows = N_PAD // NS

    pltpu.sync_copy(z_hbm.at[pl.ds(sid * rows, rows)],
                    tab_sh.at[pl.ds(sid * rows, rows)])
    plsc.subcore_barrier()
    ibase = start + wid * epw
    mbase = wid * epw

    def start_ld(c, p):
      o = c * CHUNK
      pltpu.async_copy(dst_hbm.at[pl.ds(ibase + o, CHUNK)], idx_b[p], ls[p])
      pltpu.async_copy(msg_hbm.at[pl.ds(mbase + o, CHUNK)], msg_b[p], ls[p])

    def wait_ld(p):
      pltpu.make_async_copy(dst_hbm.at[pl.ds(ibase, CHUNK)],
                            idx_b[p], ls[p]).wait()
      pltpu.make_async_copy(msg_hbm.at[pl.ds(mbase, CHUNK)],
                            msg_b[p], ls[p]).wait()

    for b in range(min(NB, nfull)):
      start_ld(b, b)

    def body(g, carry):
      for b in range(NB):
        c = NB * g + b
        wait_ld(b)
        pltpu.sync_copy(msg_b[b], tab_sh.at[idx_b[b]], add=True)

        @pl.when(c + NB < nfull)
        def _():
          start_ld(c + NB, b)

      return carry

    lax.fori_loop(0, ngrp, body, 0)
    for r in range(rem):
      wait_ld(r)
      pltpu.sync_copy(msg_b[r], tab_sh.at[idx_b[r]], add=True)
    if tail:
      idx_t = tail_refs[0]
      o = nfull * CHUNK
      pltpu.sync_copy(dst_hbm.at[pl.ds(ibase + o, tail)], idx_t)
      pltpu.sync_copy(msg_hbm.at[pl.ds(mbase + o, tail)],
                      msg_b[0].at[pl.ds(0, tail)])
      pltpu.sync_copy(msg_b[0].at[pl.ds(0, tail)], tab_sh.at[idx_t],
                      add=True)
    plsc.subcore_barrier()
    pltpu.sync_copy(tab_sh.at[pl.ds(sid * rows, rows)],
                    s_hbm.at[cid, pl.ds(sid * rows, rows)])

  return scatter_kernel


def _bn_body(x_ref, g_ref, b_ref, o_ref):
  x = x_ref[...]
  mean = jnp.mean(x, axis=0, keepdims=True)
  var = jnp.mean((x - mean) ** 2, axis=0, keepdims=True)
  o_ref[pl.ds(0, N_NODES), :] = (
      (x - mean) / jnp.sqrt(var + 1e-5) * g_ref[...] + b_ref[...])


def _mlp_body(pad_out, xi_ref, xj_ref, wa_ref, wb_ref, b1_ref, w2_ref,
              b2_ref, w3_ref, b3_ref, o_ref):
  f32, bf16 = jnp.float32, jnp.bfloat16
  h = jnp.dot(xi_ref[...].astype(bf16), wa_ref[...],
              preferred_element_type=f32)
  h += jnp.dot(xj_ref[...].astype(bf16), wb_ref[...],
               preferred_element_type=f32)
  h = jax.nn.relu(h + b1_ref[...])
  h = jax.nn.relu(jnp.dot(h.astype(bf16), w2_ref[...],
                          preferred_element_type=f32) + b2_ref[...])
  h = jnp.dot(h.astype(bf16), w3_ref[...],
              preferred_element_type=f32) + b3_ref[...]
  if pad_out:
    h = jax.nn.relu(h)
    n = h.shape[0]
    pad = jnp.concatenate(
        [jnp.ones((n, 1), f32), jnp.zeros((n, 63), f32)], axis=-1)
    h = jnp.concatenate([h, pad], axis=-1)
  o_ref[...] = h


def _mlp_call(xi, xj, wa, wb, b1, w2, b2, w3, b3, d_mid, pad_out, tile):
  ne = xi.shape[0]
  grid = (ne // tile,)
  full = lambda shape: pl.BlockSpec(shape, lambda i: (0, 0))
  return pl.pallas_call(
      functools.partial(_mlp_body, pad_out),
      grid=grid,
      in_specs=[
          pl.BlockSpec((tile, D), lambda i: (i, 0)),
          pl.BlockSpec((tile, D), lambda i: (i, 0)),
          full((D, 256)), full((D, 256)), full((1, 256)),
          full((256, 256)), full((1, 256)),
          full((256, d_mid)), full((1, d_mid)),
      ],
      out_specs=pl.BlockSpec((tile, D), lambda i: (i, 0)),
      out_shape=jax.ShapeDtypeStruct((ne, D), jnp.float32),
  )(xi, xj, wa, wb, b1, w2, b2, w3, b3)


def _combine1_body(sa_ref, sb_ref, o_ref):
  s = sa_ref[0] + sa_ref[1] + sb_ref[0] + sb_ref[1]
  cnt = s[:, 64:65]
  inv = 1.0 / jnp.maximum(cnt, 1.0)
  o_ref[...] = s * inv


def _combine2_body(sa_ref, sb_ref, ca_ref, cb_ref, o_ref):
  s = sa_ref[0] + sa_ref[1] + sb_ref[0] + sb_ref[1]
  cnt = (ca_ref[0, :, 64:65] + ca_ref[1, :, 64:65]
         + cb_ref[0, :, 64:65] + cb_ref[1, :, 64:65])
  inv = 1.0 / jnp.maximum(cnt, 1.0)
  o_ref[...] = (s * inv)[:N_NODES]


def kernel(x, edge_index, gamma, beta, W1, b1, W2, b2, W3, b3,
           W4, b4, W5, b5, W6, b6):
  src = edge_index[0].astype(jnp.int32)
  dst = edge_index[1].astype(jnp.int32)
  f32, bf16 = jnp.float32, jnp.bfloat16

  w1a = (W1[:128] - W1[128:]).astype(bf16)
  w1b = W1[128:].astype(bf16)
  zw = jnp.zeros((64, 256), f32)
  w4a = jnp.concatenate([W4[:64] - W4[64:], zw], axis=0).astype(bf16)
  w4b = jnp.concatenate([W4[64:], zw], axis=0).astype(bf16)
  w2c, w3c = W2.astype(bf16), W3.astype(bf16)
  w5c, w6c = W5.astype(bf16), W6.astype(bf16)
  b1r, b2r, b3r = b1[None, :], b2[None, :], b3[None, :]
  b4r, b5r, b6r = b4[None, :], b5[None, :], b6[None, :]
  z128 = jnp.zeros((N_PAD, D), f32)

  h = pl.pallas_call(
      _bn_body,
      out_shape=jax.ShapeDtypeStruct((N_PAD, D), f32),
  )(x, gamma[None, :], beta[None, :])

  scatters = [_make_scatter(s, n) for s, n in E_SLICES]

  def layer(tab, wa, wb, bb1, w2, bb2, w3, bb3, d_mid, pad_out):
    parts = []
    gathers = [_make_gather(s, n, N_PAD) for s, n in E_SLICES]
    pairs = [g(tab, dst, src) for g in gathers]
    for (xi, xj), sc in zip(pairs, scatters):
      m = _mlp_call(xi, xj, wa, wb, bb1, w2, bb2, w3, bb3,
                    d_mid=d_mid, pad_out=pad_out, tile=2000)
      parts.append(sc(m, dst, z128))
    return parts

  s1a, s1b = layer(h, w1a, w1b, b1r, w2c, b2r, w3c, b3r, 64, True)
  h1 = pl.pallas_call(
      _combine1_body,
      out_shape=jax.ShapeDtypeStruct((N_PAD, D), f32),
  )(s1a, s1b)

  s2a, s2b = layer(h1, w4a, w4b, b4r, w5c, b5r, w6c, b6r, 128, False)
  out = pl.pallas_call(
      _combine2_body,
      out_shape=jax.ShapeDtypeStruct((N_NODES, D), f32),
  )(s2a, s2b, s1a, s1b)
  return out
